# baseline (speedup 1.0000x reference)
import jax
import jax.numpy as jnp
from jax.experimental import pallas as pl
from jax.experimental.pallas import tpu as pltpu

_NEG = -1e30


def _bag_max_body(cats_ref, x_ref, z_ref, max_ref, acc_ref):
    b = pl.program_id(0)
    r = pl.program_id(1)
    cat = cats_ref[b]
    x = x_ref[0]
    z = z_ref[0]
    m = z == cat
    part = jnp.max(jnp.where(m, x, _NEG))

    @pl.when(r == 0)
    def _init():
        acc_ref[0] = part

    @pl.when(r > 0)
    def _acc():
        acc_ref[0] = jnp.maximum(acc_ref[0], part)

    @pl.when(r == pl.num_programs(1) - 1)
    def _emit():
        max_ref[0, 0, :] = jnp.full((128,), acc_ref[0], dtype=jnp.float32)


def _bce_body(max_ref, cats_ref, labels_ref, out_ref):
    x = max_ref[:, 0, :]
    c = cats_ref[:, 0, :]
    y = labels_ref[:, 0, :]
    valid = (c > 0) & (x > -9e29)
    r = jnp.where(valid, x, 0.0)
    per = jnp.maximum(r, 0.0) - r * y + jnp.log1p(jnp.exp(-jnp.abs(r)))
    out_ref[0, 0] = jnp.sum(per[:, 0:1]) / per.shape[0]


def kernel(pixel_logits, zone_patches, cats, labels):
    B, _, H, W = pixel_logits.shape
    logits = pixel_logits.reshape(B, H, W)

    ROWS = 128
    grid_spec = pltpu.PrefetchScalarGridSpec(
        num_scalar_prefetch=1,
        grid=(B, H // ROWS),
        in_specs=[
            pl.BlockSpec((1, ROWS, W), lambda b, r, cats: (b, r, 0)),
            pl.BlockSpec((1, ROWS, W), lambda b, r, cats: (b, r, 0)),
        ],
        out_specs=pl.BlockSpec((1, 1, 128), lambda b, r, cats: (b, 0, 0)),
        scratch_shapes=[pltpu.SMEM((1,), jnp.float32)],
    )
    bag_max = pl.pallas_call(
        _bag_max_body,
        grid_spec=grid_spec,
        out_shape=jax.ShapeDtypeStruct((B, 1, 128), jnp.float32),
    )(cats, logits, zone_patches)

    loss = pl.pallas_call(
        _bce_body,
        in_specs=[
            pl.BlockSpec((B, 1, 128), lambda: (0, 0, 0)),
            pl.BlockSpec((B, 1, 1), lambda: (0, 0, 0)),
            pl.BlockSpec((B, 1, 1), lambda: (0, 0, 0)),
        ],
        out_specs=pl.BlockSpec(memory_space=pltpu.SMEM),
        out_shape=jax.ShapeDtypeStruct((1, 1), jnp.float32),
    )(bag_max, cats.reshape(B, 1, 1), labels.reshape(B, 1, 1))

    return loss[0, 0]


# speedup vs baseline: 2.3767x; 2.3767x over previous
import jax
import jax.numpy as jnp
from jax.experimental import pallas as pl
from jax.experimental.pallas import tpu as pltpu

_NEG = -1e30


def _bag_max_body(cats_ref, x_ref, z_ref, max_ref):
    b = pl.program_id(0)
    cat = cats_ref[b]
    x = x_ref[0]
    z = z_ref[0]
    m = z == cat
    part = jnp.max(jnp.where(m, x, _NEG))
    max_ref[0, 0, :] = jnp.full((128,), part, dtype=jnp.float32)


def _bce_body(max_ref, cats_ref, labels_ref, out_ref):
    x = max_ref[:, 0, :]
    c = cats_ref[:, 0, :]
    y = labels_ref[:, 0, :]
    valid = (c > 0) & (x > -9e29)
    r = jnp.where(valid, x, 0.0)
    per = jnp.maximum(r, 0.0) - r * y + jnp.log1p(jnp.exp(-jnp.abs(r)))
    out_ref[0, 0] = jnp.sum(per[:, 0:1]) / per.shape[0]


def kernel(pixel_logits, zone_patches, cats, labels):
    B, _, H, W = pixel_logits.shape
    logits = pixel_logits.reshape(B, H, W)

    grid_spec = pltpu.PrefetchScalarGridSpec(
        num_scalar_prefetch=1,
        grid=(B,),
        in_specs=[
            pl.BlockSpec((1, H, W), lambda b, cats: (b, 0, 0)),
            pl.BlockSpec((1, H, W), lambda b, cats: (b, 0, 0)),
        ],
        out_specs=pl.BlockSpec((1, 1, 128), lambda b, cats: (b, 0, 0)),
    )
    bag_max = pl.pallas_call(
        _bag_max_body,
        grid_spec=grid_spec,
        out_shape=jax.ShapeDtypeStruct((B, 1, 128), jnp.float32),
    )(cats, logits, zone_patches)

    loss = pl.pallas_call(
        _bce_body,
        in_specs=[
            pl.BlockSpec((B, 1, 128), lambda: (0, 0, 0)),
            pl.BlockSpec((B, 1, 1), lambda: (0, 0, 0)),
            pl.BlockSpec((B, 1, 1), lambda: (0, 0, 0)),
        ],
        out_specs=pl.BlockSpec(memory_space=pltpu.SMEM),
        out_shape=jax.ShapeDtypeStruct((1, 1), jnp.float32),
    )(bag_max, cats.reshape(B, 1, 1), labels.reshape(B, 1, 1))

    return loss[0, 0]
